# Initial kernel scaffold; baseline (speedup 1.0000x reference)
import jax
import jax.numpy as jnp
from jax.experimental import pallas as pl


def kernel(features, W1, b1, g1, be1, W2, b2, g2, be2):
    raise NotImplementedError("write your pallas kernel here")


# speedup vs baseline: 12.6078x; 12.6078x over previous
import functools

import jax
import jax.numpy as jnp
from jax import lax
from jax.experimental import pallas as pl
from jax.experimental.pallas import tpu as pltpu
from jax.experimental.pallas import tpu_sc as plsc

NEG = -1e30


def _k1_body(N, T, Npad, KTOP, KP, xs_ref, xf_ref, A1_ref, V1_ref,
             idx_ref, u_ref, v_ref):
    xs = xs_ref[0]
    xf = xf_ref[0]
    inner = lax.dot_general(xs, xf, (((0,), (0,)), ((), ())),
                            preferred_element_type=jnp.float32)
    xx = jnp.sum(xf * xf, axis=0, keepdims=True)
    s = 2.0 * inner - xx
    col = lax.broadcasted_iota(jnp.int32, (T, Npad), 1)
    s = jnp.where(col < N, s, NEG)
    base = pl.program_id(0) * Npad
    cols_out = []
    for _ in range(KTOP):
        m = jnp.max(s, axis=1, keepdims=True)
        cand = jnp.where(s == m, col, Npad)
        am = jnp.min(cand, axis=1, keepdims=True)
        cols_out.append(am)
        s = jnp.where(col == am, NEG, s)
    cols_out.append(cols_out[0])
    idx_ref[0] = jnp.concatenate(cols_out, axis=1) + base
    u_ref[0] = lax.dot_general(xs, A1_ref[...], (((0,), (0,)), ((), ())),
                               preferred_element_type=jnp.float32)
    v_ref[0] = lax.dot_general(xs, V1_ref[...], (((0,), (0,)), ((), ())),
                               preferred_element_type=jnp.float32)


def _k3_body(T, KP, C, vg_ref, u_ref, W2t_ref, b1_ref, b2_ref, out_ref):
    vg = vg_ref[...].reshape(T, KP, C)
    u = u_ref[...]
    h1 = jnp.maximum(u[:, None, :] - vg + b1_ref[...][None], 0.0)
    h1f = h1.reshape(T * KP, C)
    h2 = lax.dot_general(h1f, W2t_ref[...], (((1,), (0,)), ((), ())),
                         preferred_element_type=jnp.float32)
    h2 = jnp.maximum(h2 + b2_ref[...], 0.0)
    out_ref[...] = jnp.max(h2.reshape(T, KP, C), axis=1)


def kernel(features, W1, b1, g1, be1, W2, b2, g2, be2):
    B, C, N, _ = features.shape
    KTOP = 9
    KP = 10
    T = 256
    Npad = ((N + T - 1) // T) * T
    NT = Npad // T

    x = features.reshape(B, C, N)
    xp = jnp.pad(x, ((0, 0), (0, 0), (0, Npad - N)))

    inv = 1.0 / jnp.sqrt(jnp.float32(1.0 + 1e-5))
    a1 = g1 * inv
    A1 = (a1[:, None] * (W1[:, :C] + W1[:, C:])).T
    V1 = (a1[:, None] * W1[:, C:]).T
    beta1 = (a1 * b1 + be1)[None, :]
    a2 = g2 * inv
    W2t = (a2[:, None] * W2).T
    beta2 = (a2 * b2 + be2)[None, :]

    idx, u, v = pl.pallas_call(
        functools.partial(_k1_body, N, T, Npad, KTOP, KP),
        grid=(B, NT),
        in_specs=[
            pl.BlockSpec((1, C, T), lambda b, t: (b, 0, t)),
            pl.BlockSpec((1, C, Npad), lambda b, t: (b, 0, 0)),
            pl.BlockSpec((C, C), lambda b, t: (0, 0)),
            pl.BlockSpec((C, C), lambda b, t: (0, 0)),
        ],
        out_specs=[
            pl.BlockSpec((1, T, KP), lambda b, t: (b, t, 0)),
            pl.BlockSpec((1, T, C), lambda b, t: (b, t, 0)),
            pl.BlockSpec((1, T, C), lambda b, t: (b, t, 0)),
        ],
        out_shape=[
            jax.ShapeDtypeStruct((B, Npad, KP), jnp.int32),
            jax.ShapeDtypeStruct((B, Npad, C), jnp.float32),
            jax.ShapeDtypeStruct((B, Npad, C), jnp.float32),
        ],
    )(xp, xp, A1, V1)

    TOTAL = B * Npad * KP
    NW = 32
    PER_W = TOTAL // NW
    CHUNK = 128
    NCH = PER_W // CHUNK

    v_flat = v.reshape(B * Npad, C)
    idx_flat = idx.reshape(TOTAL)

    mesh = plsc.VectorSubcoreMesh(core_axis_name="c", subcore_axis_name="s")

    @functools.partial(
        pl.kernel, mesh=mesh,
        out_type=jax.ShapeDtypeStruct((TOTAL, C), jnp.float32),
        scratch_types=[
            pltpu.VMEM((CHUNK,), jnp.int32),
            pltpu.VMEM((CHUNK, C), jnp.float32),
            pltpu.SemaphoreType.DMA,
        ],
    )
    def gather_v(table_hbm, idx_hbm, out_hbm, idx_v, rows_v, sem):
        wid = lax.axis_index("s") * 2 + lax.axis_index("c")
        wbase = wid * PER_W

        def body(i, carry):
            off = wbase + i * CHUNK
            pltpu.sync_copy(idx_hbm.at[pl.ds(off, CHUNK)], idx_v)
            pltpu.async_copy(table_hbm.at[idx_v], rows_v, sem).wait()
            pltpu.sync_copy(rows_v, out_hbm.at[pl.ds(off, CHUNK)])
            return carry

        lax.fori_loop(0, NCH, body, 0)

    vg = gather_v(v_flat, idx_flat)

    u_flat = u.reshape(B * Npad, C)
    out_flat = pl.pallas_call(
        functools.partial(_k3_body, T, KP, C),
        grid=(B * NT,),
        in_specs=[
            pl.BlockSpec((T * KP, C), lambda i: (i, 0)),
            pl.BlockSpec((T, C), lambda i: (i, 0)),
            pl.BlockSpec((C, C), lambda i: (0, 0)),
            pl.BlockSpec((1, C), lambda i: (0, 0)),
            pl.BlockSpec((1, C), lambda i: (0, 0)),
        ],
        out_specs=pl.BlockSpec((T, C), lambda i: (i, 0)),
        out_shape=jax.ShapeDtypeStruct((B * Npad, C), jnp.float32),
    )(vg, u_flat, W2t, beta1, beta2)

    out = out_flat.reshape(B, Npad, C)[:, :N].transpose(0, 2, 1)[..., None]
    return out


# speedup vs baseline: 15.7749x; 1.2512x over previous
import functools

import jax
import jax.numpy as jnp
from jax import lax
from jax.experimental import pallas as pl
from jax.experimental.pallas import tpu as pltpu
from jax.experimental.pallas import tpu_sc as plsc

NEG = -1e30


def _k1_body(N, T, Npad, KTOP, xs_ref, xf_ref, A1_ref, V1_ref,
             idx_ref, u_ref, v_ref):
    xs = xs_ref[0]
    xf = xf_ref[0]
    C = xs.shape[0]
    st = 2.0 * lax.dot_general(xf, xs, (((0,), (0,)), ((), ())),
                               preferred_element_type=jnp.float32)
    xxl = jnp.sum(xf * xf, axis=0, keepdims=True)
    xx = xxl.T
    row = lax.broadcasted_iota(jnp.int32, (Npad, T), 0)
    st = jnp.where(row < N, st - xx, NEG)
    base = pl.program_id(0) * Npad
    am0 = None
    for k in range(KTOP):
        m = jnp.max(st, axis=0, keepdims=True)
        cand = jnp.where(st == m, row, Npad)
        am = jnp.min(cand, axis=0, keepdims=True)
        am = jnp.minimum(am, N - 1)
        idx_ref[k, 0, 0, :] = (am + base)[0]
        if k < KTOP - 1:
            st = jnp.where(row == am, NEG, st)
        if k == 0:
            am0 = am
    idx_ref[KTOP, 0, 0, :] = (am0 + base)[0]
    u_ref[0] = lax.dot_general(xs, A1_ref[...], (((0,), (0,)), ((), ())),
                               preferred_element_type=jnp.float32)
    v_ref[0] = lax.dot_general(xs, V1_ref[...], (((0,), (0,)), ((), ())),
                               preferred_element_type=jnp.float32)


def _k3_body(KP, *refs):
    vg_refs = refs[:KP]
    u_ref, W2t_ref, b1_ref, b2_ref, out_ref = refs[KP:]
    u = u_ref[...]
    W2t = W2t_ref[...]
    b1 = b1_ref[...]
    b2 = b2_ref[...]
    acc = None
    for k in range(KP):
        h1 = jnp.maximum(u - vg_refs[k][0] + b1, 0.0)
        h2 = lax.dot_general(h1, W2t, (((1,), (0,)), ((), ())),
                             preferred_element_type=jnp.float32)
        h2 = jnp.maximum(h2 + b2, 0.0)
        acc = h2 if acc is None else jnp.maximum(acc, h2)
    out_ref[...] = acc


def kernel(features, W1, b1, g1, be1, W2, b2, g2, be2):
    B, C, N, _ = features.shape
    KTOP = 9
    KP = 10
    T = 256
    Npad = ((N + T - 1) // T) * T
    NT = Npad // T

    x = features.reshape(B, C, N)

    inv = 1.0 / jnp.sqrt(jnp.float32(1.0 + 1e-5))
    a1 = g1 * inv
    A1 = (a1[:, None] * (W1[:, :C] + W1[:, C:])).T
    V1 = (a1[:, None] * W1[:, C:]).T
    beta1 = (a1 * b1 + be1)[None, :]
    a2 = g2 * inv
    W2t = (a2[:, None] * W2).T
    beta2 = (a2 * b2 + be2)[None, :]

    def run_chain(xh, Bh):
        idx, u, v = pl.pallas_call(
            functools.partial(_k1_body, N, T, Npad, KTOP),
            grid=(Bh, NT),
            in_specs=[
                pl.BlockSpec((1, C, T), lambda b, t: (b, 0, t)),
                pl.BlockSpec((1, C, Npad), lambda b, t: (b, 0, 0)),
                pl.BlockSpec((C, C), lambda b, t: (0, 0)),
                pl.BlockSpec((C, C), lambda b, t: (0, 0)),
            ],
            out_specs=[
                pl.BlockSpec((KP, 1, 1, T),
                             lambda b, t: (0, b * NT + t, 0, 0)),
                pl.BlockSpec((1, T, C), lambda b, t: (b, t, 0)),
                pl.BlockSpec((1, T, C), lambda b, t: (b, t, 0)),
            ],
            out_shape=[
                jax.ShapeDtypeStruct((KP, Bh * NT, 1, T), jnp.int32),
                jax.ShapeDtypeStruct((Bh, Npad, C), jnp.float32),
                jax.ShapeDtypeStruct((Bh, Npad, C), jnp.float32),
            ],
        )(xh, xh, A1, V1)

        M = Bh * Npad
        TOTAL = M * KP
        NW = 32
        PER_W = TOTAL // NW
        CHUNK = 128
        NCH = PER_W // CHUNK

        v_flat = v.reshape(M, C)
        idx_flat = idx.reshape(TOTAL)

        mesh = plsc.VectorSubcoreMesh(core_axis_name="c",
                                      subcore_axis_name="s")

        NBUF = 4 if NCH % 4 == 0 else 2
        NOUTER = NCH // NBUF

        @functools.partial(
            pl.kernel, mesh=mesh,
            out_type=jax.ShapeDtypeStruct((TOTAL, C), jnp.float32),
            scratch_types=[
                pltpu.VMEM((PER_W,), jnp.int32),
                pltpu.VMEM((NBUF, CHUNK, C), jnp.float32),
            ] + [pltpu.SemaphoreType.DMA] * (2 * NBUF),
        )
        def gather_v(table_hbm, idx_hbm, out_hbm, idx_v, rows_v, *sems):
            gsem = sems[:NBUF]
            ssem = sems[NBUF:]
            wid = lax.axis_index("s") * 2 + lax.axis_index("c")
            wbase = wid * PER_W
            pltpu.sync_copy(idx_hbm.at[pl.ds(wbase, PER_W)], idx_v)

            def body(j, carry):
                for b in range(NBUF):
                    i = j * NBUF + b
                    off = wbase + i * CHUNK

                    @pl.when(j > 0)
                    def _():
                        pltpu.make_async_copy(
                            rows_v.at[b],
                            out_hbm.at[pl.ds(off - NBUF * CHUNK, CHUNK)],
                            ssem[b]).wait()

                    pltpu.async_copy(
                        table_hbm.at[idx_v.at[pl.ds(i * CHUNK, CHUNK)]],
                        rows_v.at[b], gsem[b]).wait()
                    pltpu.async_copy(rows_v.at[b],
                                     out_hbm.at[pl.ds(off, CHUNK)], ssem[b])
                return carry

            lax.fori_loop(0, NOUTER, body, 0)
            for b in range(NBUF):
                i = (NOUTER - 1) * NBUF + b
                pltpu.make_async_copy(
                    rows_v.at[b],
                    out_hbm.at[pl.ds(wbase + i * CHUNK, CHUNK)],
                    ssem[b]).wait()

        vg = gather_v(v_flat, idx_flat).reshape(KP, M, C)

        u_flat = u.reshape(M, C)
        vg_specs = [
            pl.BlockSpec((1, T, C),
                         functools.partial(lambda k, i: (k, i, 0), k))
            for k in range(KP)
        ]
        return pl.pallas_call(
            functools.partial(_k3_body, KP),
            grid=(Bh * NT,),
            in_specs=vg_specs + [
                pl.BlockSpec((T, C), lambda i: (i, 0)),
                pl.BlockSpec((C, C), lambda i: (0, 0)),
                pl.BlockSpec((1, C), lambda i: (0, 0)),
                pl.BlockSpec((1, C), lambda i: (0, 0)),
            ],
            out_specs=pl.BlockSpec((T, C), lambda i: (i, 0)),
            out_shape=jax.ShapeDtypeStruct((M, C), jnp.float32),
        )(*([vg] * KP), u_flat, W2t, beta1, beta2)

    NSPLIT = 4
    Bh = B // NSPLIT
    out_flat = jnp.concatenate(
        [run_chain(x[i * Bh:(i + 1) * Bh], Bh) for i in range(NSPLIT)],
        axis=0)

    out = out_flat.reshape(B, Npad, C)[:, :N].transpose(0, 2, 1)[..., None]
    return out
